# baseline (speedup 1.0000x reference)
import jax
import jax.numpy as jnp
from jax.experimental import pallas as pl
from jax.experimental.pallas import tpu as pltpu

_HW = 7
_IMG = 224
_NLM = 8
_BB = 8


def _roi_kernel(feat_ref, xs_ref, ys_ref, out_ref):
    feat = feat_ref[...]
    xs = xs_ref[...]
    ys = ys_ref[...]
    x1 = jnp.clip((xs * _HW) // _IMG - 1, 0, _HW - 2)
    y1 = jnp.clip((ys * _HW) // _IMG - 1, 0, _HW - 2)
    pstar = x1 * _HW + y1

    h = jnp.maximum(feat[:, :, :42], feat[:, :, 7:49])
    pool = jnp.maximum(h[:, :, :41], h[:, :, 1:42])

    c = feat.shape[1]
    p_iota = jax.lax.broadcasted_iota(jnp.int32, (1, 1, 41), 2)
    for l in range(_NLM):
        psel = pstar[:, l].reshape(_BB, 1, 1)
        sel = jnp.where(p_iota == psel, pool, -jnp.inf)
        out_ref[:, l * c:(l + 1) * c] = jnp.max(sel, axis=-1)


def kernel(features, landmarks):
    B, C, H, W = features.shape
    feat = features.reshape(B, C, H * W)
    xs = landmarks[:, 0::2]
    ys = landmarks[:, 1::2]
    return pl.pallas_call(
        _roi_kernel,
        out_shape=jax.ShapeDtypeStruct((B, _NLM * C), features.dtype),
        grid=(B // _BB,),
        in_specs=[
            pl.BlockSpec((_BB, C, H * W), lambda i: (i, 0, 0)),
            pl.BlockSpec((_BB, _NLM), lambda i: (i, 0)),
            pl.BlockSpec((_BB, _NLM), lambda i: (i, 0)),
        ],
        out_specs=pl.BlockSpec((_BB, _NLM * C), lambda i: (i, 0)),
        compiler_params=pltpu.CompilerParams(
            dimension_semantics=("parallel",),
        ),
    )(feat, xs, ys)


# speedup vs baseline: 1.6690x; 1.6690x over previous
import jax
import jax.numpy as jnp
from jax.experimental import pallas as pl
from jax.experimental.pallas import tpu as pltpu

_HW = 7
_IMG = 224
_NLM = 8
_BB = 8


def _roi_kernel(feat_ref, xs_ref, ys_ref, out_ref):
    feat = feat_ref[...]
    xs = xs_ref[...]
    ys = ys_ref[...]
    x1 = jnp.clip((xs * _HW) // _IMG - 1, 0, _HW - 2)
    y1 = jnp.clip((ys * _HW) // _IMG - 1, 0, _HW - 2)
    pstar = x1 * _HW + y1

    h = jnp.maximum(feat[:, :, :42], feat[:, :, 7:49])
    pool = jnp.maximum(h[:, :, :41], h[:, :, 1:42])

    idx = jnp.broadcast_to(pstar[:, None, :], pool.shape[:2] + (_NLM,))
    out_ref[...] = jnp.take_along_axis(pool, idx, axis=2)


def kernel(features, landmarks):
    B, C, H, W = features.shape
    feat = features.reshape(B, C, H * W)
    xs = landmarks[:, 0::2]
    ys = landmarks[:, 1::2]
    out = pl.pallas_call(
        _roi_kernel,
        out_shape=jax.ShapeDtypeStruct((B, C, _NLM), features.dtype),
        grid=(B // _BB,),
        in_specs=[
            pl.BlockSpec((_BB, C, H * W), lambda i: (i, 0, 0)),
            pl.BlockSpec((_BB, _NLM), lambda i: (i, 0)),
            pl.BlockSpec((_BB, _NLM), lambda i: (i, 0)),
        ],
        out_specs=pl.BlockSpec((_BB, C, _NLM), lambda i: (i, 0, 0)),
        compiler_params=pltpu.CompilerParams(
            dimension_semantics=("parallel",),
        ),
    )(feat, xs, ys)
    return out.transpose(0, 2, 1).reshape(B, _NLM * C)


# speedup vs baseline: 2.0436x; 1.2245x over previous
import jax
import jax.numpy as jnp
from jax.experimental import pallas as pl
from jax.experimental.pallas import tpu as pltpu

_HW = 7
_IMG = 224
_NLM = 8
_BB = 8


_CCH = 512


def _roi_kernel(feat_ref, lm_ref, out_ref):
    lm = lm_ref[...]
    st = jnp.clip((lm * _HW) // _IMG - 1, 0, _HW - 2)
    even = 2 * jax.lax.broadcasted_iota(jnp.int32, (_BB, _NLM), 1)
    x1 = jnp.take_along_axis(st, even, axis=1)
    y1 = jnp.take_along_axis(st, even + 1, axis=1)
    pstar = x1 * _HW + y1
    idx = jnp.broadcast_to(pstar[:, None, :], (_BB, _CCH, _NLM))

    c = feat_ref.shape[1]
    for c0 in range(0, c, _CCH):
        feat = feat_ref[:, c0:c0 + _CCH, :]
        h = jnp.maximum(feat[:, :, :42], feat[:, :, 7:49])
        pool = jnp.maximum(h[:, :, :41], h[:, :, 1:42])
        g = jnp.take_along_axis(pool, idx, axis=2)
        out_ref[:, :, c0:c0 + _CCH] = jnp.swapaxes(g, 1, 2)


def kernel(features, landmarks):
    B, C, H, W = features.shape
    feat = features.reshape(B, C, H * W)
    out = pl.pallas_call(
        _roi_kernel,
        out_shape=jax.ShapeDtypeStruct((B, _NLM, C), features.dtype),
        grid=(B // _BB,),
        in_specs=[
            pl.BlockSpec((_BB, C, H * W), lambda i: (i, 0, 0)),
            pl.BlockSpec((_BB, 2 * _NLM), lambda i: (i, 0)),
        ],
        out_specs=pl.BlockSpec((_BB, _NLM, C), lambda i: (i, 0, 0)),
        compiler_params=pltpu.CompilerParams(
            dimension_semantics=("parallel",),
        ),
    )(feat, landmarks)
    return out.reshape(B, _NLM * C)


# speedup vs baseline: 10.5787x; 5.1764x over previous
import jax
import jax.numpy as jnp
from jax.experimental import pallas as pl
from jax.experimental.pallas import tpu as pltpu

_HW = 7
_NLM = 8
_BB = 8
_NP = 6


def _roi_kernel(lm_ref, feat_ref, out_ref, pool_ref):
    c = feat_ref.shape[3]
    x = [[feat_ref[h, w] for w in range(_HW)] for h in range(_HW)]
    hm = [[jnp.maximum(x[h][w], x[h][w + 1]) for w in range(_NP)]
          for h in range(_HW)]
    for h in range(_NP):
        for w in range(_NP):
            pool_ref[h * _NP + w] = jnp.maximum(hm[h][w], hm[h + 1][w])

    b0 = pl.program_id(0) * _BB
    for b in range(_BB):
        for l in range(_NLM):
            p = lm_ref[(b0 + b) * _NLM + l]
            out_ref[b, l * c:(l + 1) * c] = pool_ref[p, b]


def kernel(features, landmarks):
    B, C, H, W = features.shape
    feat = jnp.transpose(features, (2, 3, 0, 1))
    st = jnp.clip((landmarks >> 5) - 1, 0, H - 2).reshape(B, _NLM, 2)
    pidx = (st[:, :, 0] * _NP + st[:, :, 1]).reshape(B * _NLM)
    return pl.pallas_call(
        _roi_kernel,
        out_shape=jax.ShapeDtypeStruct((B, _NLM * C), features.dtype),
        grid_spec=pltpu.PrefetchScalarGridSpec(
            num_scalar_prefetch=1,
            grid=(B // _BB,),
            in_specs=[
                pl.BlockSpec((H, W, _BB, C), lambda i, lm: (0, 0, i, 0)),
            ],
            out_specs=pl.BlockSpec((_BB, _NLM * C), lambda i, lm: (i, 0)),
            scratch_shapes=[pltpu.VMEM((_NP * _NP, _BB, C), jnp.float32)],
        ),
        compiler_params=pltpu.CompilerParams(
            dimension_semantics=("parallel",),
        ),
    )(pidx, feat)


# speedup vs baseline: 19.0794x; 1.8036x over previous
import jax
import jax.numpy as jnp
from jax.experimental import pallas as pl
from jax.experimental.pallas import tpu as pltpu

_HW = 7
_NLM = 8
_BB = 32
_NP = 6


def _roi_kernel(lm_ref, feat_ref, out_ref, pool_ref):
    c = feat_ref.shape[3]
    x = [[feat_ref[h, w] for w in range(_HW)] for h in range(_HW)]
    hm = [[jnp.maximum(x[h][w], x[h][w + 1]) for w in range(_NP)]
          for h in range(_HW)]
    for h in range(_NP):
        for w in range(_NP):
            pool_ref[h * _NP + w] = jnp.maximum(hm[h][w], hm[h + 1][w])

    b0 = pl.program_id(0) * _BB
    for b in range(_BB):
        for l in range(_NLM):
            p = lm_ref[(b0 + b) * _NLM + l]
            out_ref[b, l * c:(l + 1) * c] = pool_ref[p, b]


def kernel(features, landmarks):
    B, C, H, W = features.shape
    feat = jnp.transpose(features, (2, 3, 0, 1))
    st = jnp.clip((landmarks >> 5) - 1, 0, H - 2).reshape(B, _NLM, 2)
    pidx = (st[:, :, 0] * _NP + st[:, :, 1]).reshape(B * _NLM)
    return pl.pallas_call(
        _roi_kernel,
        out_shape=jax.ShapeDtypeStruct((B, _NLM * C), features.dtype),
        grid_spec=pltpu.PrefetchScalarGridSpec(
            num_scalar_prefetch=1,
            grid=(B // _BB,),
            in_specs=[
                pl.BlockSpec((H, W, _BB, C), lambda i, lm: (0, 0, i, 0)),
            ],
            out_specs=pl.BlockSpec((_BB, _NLM * C), lambda i, lm: (i, 0)),
            scratch_shapes=[pltpu.VMEM((_NP * _NP, _BB, C), jnp.float32)],
        ),
        compiler_params=pltpu.CompilerParams(
            dimension_semantics=("parallel",),
        ),
    )(pidx, feat)


# speedup vs baseline: 21.3717x; 1.1201x over previous
import jax
import jax.numpy as jnp
from jax.experimental import pallas as pl
from jax.experimental.pallas import tpu as pltpu

_HW = 7
_NLM = 8
_BB = 64
_NP = 6


def _roi_kernel(lm_ref, feat_ref, out_ref, pool_ref):
    c = feat_ref.shape[3]
    x = [[feat_ref[h, w] for w in range(_HW)] for h in range(_HW)]
    hm = [[jnp.maximum(x[h][w], x[h][w + 1]) for w in range(_NP)]
          for h in range(_HW)]
    for h in range(_NP):
        for w in range(_NP):
            pool_ref[h * _NP + w] = jnp.maximum(hm[h][w], hm[h + 1][w])

    b0 = pl.program_id(0) * _BB
    for b in range(_BB):
        for l in range(_NLM):
            p = lm_ref[(b0 + b) * _NLM + l]
            out_ref[b, l * c:(l + 1) * c] = pool_ref[p, b]


def kernel(features, landmarks):
    B, C, H, W = features.shape
    feat = jnp.transpose(features, (2, 3, 0, 1))
    st = jnp.clip((landmarks >> 5) - 1, 0, H - 2).reshape(B, _NLM, 2)
    pidx = (st[:, :, 0] * _NP + st[:, :, 1]).reshape(B * _NLM)
    return pl.pallas_call(
        _roi_kernel,
        out_shape=jax.ShapeDtypeStruct((B, _NLM * C), features.dtype),
        grid_spec=pltpu.PrefetchScalarGridSpec(
            num_scalar_prefetch=1,
            grid=(B // _BB,),
            in_specs=[
                pl.BlockSpec((H, W, _BB, C), lambda i, lm: (0, 0, i, 0)),
            ],
            out_specs=pl.BlockSpec((_BB, _NLM * C), lambda i, lm: (i, 0)),
            scratch_shapes=[pltpu.VMEM((_NP * _NP, _BB, C), jnp.float32)],
        ),
        compiler_params=pltpu.CompilerParams(
            dimension_semantics=("parallel",),
        ),
    )(pidx, feat)


# speedup vs baseline: 22.9429x; 1.0735x over previous
import jax
import jax.numpy as jnp
from jax.experimental import pallas as pl
from jax.experimental.pallas import tpu as pltpu

_HW = 7
_NLM = 8
_BB = 128
_NP = 6


def _roi_kernel(lm_ref, feat_ref, out_ref, pool_ref):
    c = feat_ref.shape[3]
    x = [[feat_ref[h, w] for w in range(_HW)] for h in range(_HW)]
    hm = [[jnp.maximum(x[h][w], x[h][w + 1]) for w in range(_NP)]
          for h in range(_HW)]
    for h in range(_NP):
        for w in range(_NP):
            pool_ref[h * _NP + w] = jnp.maximum(hm[h][w], hm[h + 1][w])

    b0 = pl.program_id(0) * _BB
    for b in range(_BB):
        for l in range(_NLM):
            p = lm_ref[(b0 + b) * _NLM + l]
            out_ref[b, l * c:(l + 1) * c] = pool_ref[p, b]


def kernel(features, landmarks):
    B, C, H, W = features.shape
    feat = jnp.transpose(features, (2, 3, 0, 1))
    st = jnp.clip((landmarks >> 5) - 1, 0, H - 2).reshape(B, _NLM, 2)
    pidx = (st[:, :, 0] * _NP + st[:, :, 1]).reshape(B * _NLM)
    return pl.pallas_call(
        _roi_kernel,
        out_shape=jax.ShapeDtypeStruct((B, _NLM * C), features.dtype),
        grid_spec=pltpu.PrefetchScalarGridSpec(
            num_scalar_prefetch=1,
            grid=(B // _BB,),
            in_specs=[
                pl.BlockSpec((H, W, _BB, C), lambda i, lm: (0, 0, i, 0)),
            ],
            out_specs=pl.BlockSpec((_BB, _NLM * C), lambda i, lm: (i, 0)),
            scratch_shapes=[pltpu.VMEM((_NP * _NP, _BB, C), jnp.float32)],
        ),
        compiler_params=pltpu.CompilerParams(
            dimension_semantics=("parallel",),
        ),
    )(pidx, feat)
